# Initial kernel scaffold; baseline (speedup 1.0000x reference)
import jax
import jax.numpy as jnp
from jax.experimental import pallas as pl


def kernel(x, A):
    raise NotImplementedError("write your pallas kernel here")


# speedup vs baseline: 1.0360x; 1.0360x over previous
import jax
import jax.numpy as jnp
from jax.experimental import pallas as pl
from jax.experimental.pallas import tpu as pltpu

_BB = 64


def _nconv_bf16_kernel(a_ref, x_ref, o_ref):
    a = a_ref[...]
    for b in range(x_ref.shape[0]):
        xb = x_ref[b].astype(jnp.bfloat16)
        o_ref[b] = jnp.dot(a, xb, preferred_element_type=jnp.float32)


def kernel(x, A):
    N, C, W, L = x.shape
    V = A.shape[0]
    B = N * C
    x3 = x.reshape(B, W, L)
    a_bf = A.astype(jnp.bfloat16)

    bb = _BB
    grid = (B // bb,)
    flops = 2 * B * V * W * L
    bytes_accessed = (B * W * L + B * V * L) * 4 + V * W * 2

    out = pl.pallas_call(
        _nconv_bf16_kernel,
        out_shape=jax.ShapeDtypeStruct((B, V, L), jnp.float32),
        grid=grid,
        in_specs=[
            pl.BlockSpec((V, W), lambda i: (0, 0)),
            pl.BlockSpec((bb, W, L), lambda i: (i, 0, 0)),
        ],
        out_specs=pl.BlockSpec((bb, V, L), lambda i: (i, 0, 0)),
        compiler_params=pltpu.CompilerParams(
            dimension_semantics=("parallel",),
            vmem_limit_bytes=100 * 1024 * 1024,
        ),
        cost_estimate=pl.CostEstimate(
            flops=flops, transcendentals=0, bytes_accessed=bytes_accessed),
    )(a_bf, x3)
    return out.reshape(N, C, V, L)
